# Initial kernel scaffold; baseline (speedup 1.0000x reference)
import jax
import jax.numpy as jnp
from jax.experimental import pallas as pl


def kernel(coords, scales, quats, opacity, sh, viewmat, cam_pos):
    raise NotImplementedError("write your pallas kernel here")


# speedup vs baseline: 2.0975x; 2.0975x over previous
import jax
import jax.numpy as jnp
from jax.experimental import pallas as pl

WIMG = 256.0
FX = 220.0
FY = 220.0
NT = 16
TS = 16
KSEL = 64

C0 = 0.28209479177387814
C1 = 0.4886025119029199
C2 = (1.0925484305920792, -1.0925484305920792, 0.31539156525252005,
      -1.0925484305920792, 0.5462742152960396)
C3 = (-0.5900435899266435, 2.890611442640554, -0.4570457994644658,
      0.3731763325901154, -0.4570457994644658, 1.445305721320277,
      -0.5900435899266435)

_B = 2048


def _feat_body(inp_ref, cam_ref, out_ref):
    f = inp_ref[...]
    cam = cam_ref[...]

    def cs(k):
        return cam[0:1, k:k + 1]

    x = f[0:1]; y = f[1:2]; z = f[2:3]
    sx = f[3:4]; sy = f[4:5]; sz = f[5:6]
    qr = f[6:7]; qx = f[7:8]; qy = f[8:9]; qz = f[9:10]
    op = f[10:11]; vld = f[11:12]

    w00 = cs(0); w01 = cs(1); w02 = cs(2)
    w10 = cs(3); w11 = cs(4); w12 = cs(5)
    w20 = cs(6); w21 = cs(7); w22 = cs(8)
    t0 = cs(9); t1 = cs(10); t2 = cs(11)
    cx = cs(12); cy = cs(13); cz = cs(14)

    def bf(v):
        return jax.lax.convert_element_type(
            jax.lax.convert_element_type(v, jnp.bfloat16), jnp.float32)

    xb = bf(x); yb = bf(y); zb = bf(z)
    w00b = bf(w00); w01b = bf(w01); w02b = bf(w02)
    w10b = bf(w10); w11b = bf(w11); w12b = bf(w12)
    w20b = bf(w20); w21b = bf(w21); w22b = bf(w22)
    tx = (xb * w00b + yb * w01b) + zb * w02b + t0
    ty = (xb * w10b + yb * w11b) + zb * w12b + t1
    tz = (xb * w20b + yb * w21b) + zb * w22b + t2
    frontf = jnp.where((tz > 0.2) & (vld > 0.5), 1.0, 0.0)
    tzs = jnp.maximum(tz, 0.2)
    px = FX * tx / tzs + WIMG / 2.0
    py = FY * ty / tzs + WIMG / 2.0

    qn = jnp.sqrt(qr * qr + qx * qx + qy * qy + qz * qz) + 1e-8
    r_ = qr / qn; xq = qx / qn; yq = qy / qn; zq = qz / qn
    r00 = 1.0 - 2.0 * (yq * yq + zq * zq)
    r01 = 2.0 * (xq * yq - r_ * zq)
    r02 = 2.0 * (xq * zq + r_ * yq)
    r10 = 2.0 * (xq * yq + r_ * zq)
    r11 = 1.0 - 2.0 * (xq * xq + zq * zq)
    r12 = 2.0 * (yq * zq - r_ * xq)
    r20 = 2.0 * (xq * zq - r_ * yq)
    r21 = 2.0 * (yq * zq + r_ * xq)
    r22 = 1.0 - 2.0 * (xq * xq + yq * yq)
    m00 = bf(r00 * sx); m01 = bf(r01 * sy); m02 = bf(r02 * sz)
    m10 = bf(r10 * sx); m11 = bf(r11 * sy); m12 = bf(r12 * sz)
    m20 = bf(r20 * sx); m21 = bf(r21 * sy); m22 = bf(r22 * sz)
    c00 = (m00 * m00 + m01 * m01) + m02 * m02
    c01 = (m00 * m10 + m01 * m11) + m02 * m12
    c02 = (m00 * m20 + m01 * m21) + m02 * m22
    c10 = (m10 * m00 + m11 * m01) + m12 * m02
    c11 = (m10 * m10 + m11 * m11) + m12 * m12
    c12 = (m10 * m20 + m11 * m21) + m12 * m22
    c20 = (m20 * m00 + m21 * m01) + m22 * m02
    c21 = (m20 * m10 + m21 * m11) + m22 * m12
    c22 = (m20 * m20 + m21 * m21) + m22 * m22

    tzs2 = tzs * tzs
    j00 = bf(FX / tzs); j02 = bf(-FX * tx / tzs2)
    j11 = bf(FY / tzs); j12 = bf(-FY * ty / tzs2)
    u0 = j00 * w00b + j02 * w20b
    u1 = j00 * w01b + j02 * w21b
    u2 = j00 * w02b + j02 * w22b
    v0 = j11 * w10b + j12 * w20b
    v1 = j11 * w11b + j12 * w21b
    v2 = j11 * w12b + j12 * w22b
    ub0 = bf(u0); ub1 = bf(u1); ub2 = bf(u2)
    vb0 = bf(v0); vb1 = bf(v1); vb2 = bf(v2)
    c00b = bf(c00); c01b = bf(c01); c02b = bf(c02)
    c10b = bf(c10); c11b = bf(c11); c12b = bf(c12)
    c20b = bf(c20); c21b = bf(c21); c22b = bf(c22)
    p00 = bf((ub0 * c00b + ub1 * c10b) + ub2 * c20b)
    p01 = bf((ub0 * c01b + ub1 * c11b) + ub2 * c21b)
    p02 = bf((ub0 * c02b + ub1 * c12b) + ub2 * c22b)
    p10 = bf((vb0 * c00b + vb1 * c10b) + vb2 * c20b)
    p11 = bf((vb0 * c01b + vb1 * c11b) + vb2 * c21b)
    p12 = bf((vb0 * c02b + vb1 * c12b) + vb2 * c22b)
    a = (p00 * ub0 + p01 * ub1) + p02 * ub2 + 0.3
    b = (p00 * vb0 + p01 * vb1) + p02 * vb2
    b2 = (p10 * ub0 + p11 * ub1) + p12 * ub2
    d = (p10 * vb0 + p11 * vb1) + p12 * vb2 + 0.3
    trace = a + d
    det = a * d - b * b2
    term2 = 0.5 * jnp.sqrt(jnp.maximum(trace * trace - 4.0 * det, 1e-12))
    term1 = 0.5 * trace
    radius = 3.0 * jnp.sqrt(jnp.maximum(term1 - term2, term1 + term2))
    i00 = d / det
    i01s = (-b / det) + (-b2 / det)
    i11 = a / det

    dxx = x - cx; dyy = y - cy; dzz = z - cz
    dn = jnp.sqrt(dxx * dxx + dyy * dyy + dzz * dzz) + 1e-8
    dx = dxx / dn; dy = dyy / dn; dz = dzz / dn
    xx = dx * dx; yy = dy * dy; zz = dz * dz
    xy = dx * dy; yz = dy * dz; xz = dx * dz
    basis = (None, -C1 * dy, C1 * dz, -C1 * dx,
             C2[0] * xy, C2[1] * yz, C2[2] * (2.0 * zz - xx - yy),
             C2[3] * xz, C2[4] * (xx - yy),
             C3[0] * dy * (3.0 * xx - yy), C3[1] * xy * dz,
             C3[2] * dy * (4.0 * zz - xx - yy),
             C3[3] * dz * (2.0 * zz - 3.0 * xx - 3.0 * yy),
             C3[4] * dx * (4.0 * zz - xx - yy),
             C3[5] * dz * (xx - yy), C3[6] * dx * (xx - 3.0 * yy))
    cols = []
    for c in range(3):
        acc = C0 * f[12 + c:13 + c]
        for l in range(1, 16):
            acc = acc + basis[l] * f[12 + 3 * l + c:13 + 3 * l + c]
        cols.append(jnp.maximum(acc + 0.5, 0.0))

    out = jnp.concatenate([
        px, py, tz, radius, i00, i01s, i11, op,
        cols[0], cols[1], cols[2], frontf,
        px - radius, px + radius, py - radius, py + radius,
    ], axis=0)
    out_ref[...] = out


def _blend_body(pack_ref, out_ref):
    f = pack_ref[0]
    g_px = f[:, 0:1]; g_py = f[:, 1:2]
    i00 = f[:, 2:3]; i01s = f[:, 3:4]; i11 = f[:, 4:5]
    g_op = f[:, 5:6]; validc = f[:, 6:7]
    t = pl.program_id(0)
    txf = jax.lax.convert_element_type((t // NT) * TS, jnp.float32)
    tyf = jax.lax.convert_element_type((t % NT) * TS, jnp.float32)
    ii = jax.lax.broadcasted_iota(jnp.int32, (1, TS * TS), 1)
    xi = jax.lax.convert_element_type(ii // TS, jnp.float32)
    yi = jax.lax.convert_element_type(ii % TS, jnp.float32)
    xs = xi + txf + 0.5
    ys = yi + tyf + 0.5
    dx = xs - g_px
    dy = ys - g_py
    power = -0.5 * (i00 * dx * dx + i11 * dy * dy + i01s * dx * dy)
    prob = jnp.exp(jnp.minimum(power, 0.0))
    alpha = jnp.clip(g_op * prob, 0.01, 0.99) * validc
    om = 1.0 - alpha
    e = jnp.concatenate([jnp.ones((1, TS * TS), jnp.float32), om[:KSEL - 1]], axis=0)
    s = 1
    while s < KSEL:
        e = e * jnp.concatenate(
            [jnp.ones((s, TS * TS), jnp.float32), e[:KSEL - s]], axis=0)
        s *= 2
    contrib = alpha * e
    for c in range(3):
        cc = f[:, 8 + c:9 + c]
        out_ref[0, c:c + 1, :] = jnp.sum(contrib * cc, axis=0, keepdims=True)


def _features(coords, scales, quats, opacity, sh, viewmat, cam_pos):
    n = coords.shape[0]
    npad = ((n + _B - 1) // _B) * _B
    pad = npad - n

    def padt(arr2):
        return jnp.pad(arr2, ((0, 0), (0, pad)))

    inp = jnp.concatenate([
        padt(coords.T.astype(jnp.float32)),
        padt(scales.T.astype(jnp.float32)),
        padt(quats.T.astype(jnp.float32)),
        padt(opacity[None].astype(jnp.float32)),
        jnp.pad(jnp.ones((1, n), jnp.float32), ((0, 0), (0, pad))),
        padt(sh.astype(jnp.float32).transpose(1, 2, 0).reshape(48, n)),
        jnp.zeros((4, npad), jnp.float32),
    ], axis=0)
    wm = viewmat[:3, :3].astype(jnp.float32)
    camv = jnp.zeros((8, 128), jnp.float32).at[0, :15].set(
        jnp.concatenate([wm.reshape(9), viewmat[:3, 3].astype(jnp.float32),
                         cam_pos.astype(jnp.float32)]))

    feats = pl.pallas_call(
        _feat_body,
        grid=(npad // _B,),
        in_specs=[
            pl.BlockSpec((64, _B), lambda i: (0, i)),
            pl.BlockSpec((8, 128), lambda i: (0, 0)),
        ],
        out_specs=pl.BlockSpec((16, _B), lambda i: (0, i)),
        out_shape=jax.ShapeDtypeStruct((16, npad), jnp.float32),
    )(inp, camv)
    return feats


def kernel(coords, scales, quats, opacity, sh, viewmat, cam_pos):
    feats = _features(coords, scales, quats, opacity, sh, viewmat, cam_pos)
    depth = feats[2]
    frontf = feats[11]
    xlo = feats[12]; xhi = feats[13]; ylo = feats[14]; yhi = feats[15]
    tids = jnp.arange(NT * NT)
    left = ((tids // NT) * TS).astype(jnp.float32)
    top = ((tids % NT) * TS).astype(jnp.float32)
    inter = ((xhi[None, :] >= left[:, None]) & (xlo[None, :] <= left[:, None] + TS)
             & (yhi[None, :] >= top[:, None]) & (ylo[None, :] <= top[:, None] + TS)
             & (frontf[None, :] > 0.5))
    score = jnp.where(inter, -depth[None, :], -1e10)
    sc, idx = jax.lax.top_k(score, KSEL)
    validf = (sc > -1e9).astype(jnp.float32)

    g = feats[:, idx.reshape(-1)].reshape(16, NT * NT, KSEL)
    pack = jnp.stack([
        g[0], g[1], g[4], g[5], g[6], g[7], validf, jnp.zeros_like(validf),
        g[8], g[9], g[10], jnp.zeros_like(validf),
    ], axis=-1)
    pack = jnp.pad(pack, ((0, 0), (0, 0), (0, 4)))

    out = pl.pallas_call(
        _blend_body,
        grid=(NT * NT,),
        in_specs=[pl.BlockSpec((1, KSEL, 16), lambda i: (i, 0, 0))],
        out_specs=pl.BlockSpec((1, 3, TS * TS), lambda i: (i, 0, 0)),
        out_shape=jax.ShapeDtypeStruct((NT * NT, 3, TS * TS), jnp.float32),
    )(pack)

    img = out.reshape(NT, NT, 3, TS, TS).transpose(0, 3, 1, 4, 2)
    return img.reshape(int(WIMG), int(WIMG), 3)
